# baseline (speedup 1.0000x reference)
import functools

import jax
import jax.numpy as jnp
from jax import lax
from jax.experimental import pallas as pl
from jax.experimental.pallas import tpu as pltpu
from jax.experimental.pallas import tpu_sc as plsc

B, H, W, C = 32, 512, 512, 3
NBINS = 256
HW = H * W
_W0, _W1, _W2 = 0.29882812, 0.5859375, 0.11376953

_NC, _NS = 2, 16
_G = 8192
_IN = _G * 3
_NCH = HW // _G


def _b16(v):
    u = plsc.bitcast(v, jnp.int32)
    lsb = lax.shift_right_logical(u, 16) & 1
    r = (u + 0x7FFF + lsb) & jnp.int32(-65536)
    return plsc.bitcast(r, jnp.float32)


def _sc_gray_body(x_hbm, gray_hbm, gmn_hbm, gmx_hbm,
                  inb0, inb1, gout0, gout1, v16a, v16b,
                  sem_in0, sem_in1, sem_out0, sem_out1):
    wid = lax.axis_index("s") * _NC + lax.axis_index("c")
    base_in = wid * (HW * C)
    base_g = wid * HW

    iota = lax.broadcasted_iota(jnp.int32, (16,), 0)
    iota3 = iota * 3
    w0 = jnp.float32(_W0)
    w1 = jnp.float32(_W1)
    w2 = jnp.float32(_W2)

    inbs = (inb0, inb1)
    gouts = (gout0, gout1)
    sins = (sem_in0, sem_in1)
    souts = (sem_out0, sem_out1)

    pltpu.async_copy(x_hbm.at[pl.ds(base_in, _IN)], inb0, sem_in0)
    pltpu.async_copy(x_hbm.at[pl.ds(base_in + _IN, _IN)], inb1, sem_in1)

    mn0 = jnp.full((16,), jnp.inf, jnp.float32)
    mx0 = jnp.full((16,), -jnp.inf, jnp.float32)

    @pl.loop(0, _NCH, step=2, init_carry=(mn0, mx0))
    def p1(ci, carry):
        mn, mx = carry
        for b in range(2):
            cur = ci + b
            inb, gout, sin, sout = inbs[b], gouts[b], sins[b], souts[b]
            pltpu.make_async_copy(
                x_hbm.at[pl.ds(base_in + cur * _IN, _IN)], inb, sin).wait()

            @pl.when(cur >= 2)
            def _():
                pltpu.make_async_copy(
                    gout, gray_hbm.at[pl.ds(base_g + (cur - 2) * _G, _G)],
                    sout).wait()

            def ibody(k, c, inb=inb, gout=gout):
                mn_, mx_ = c
                idx0 = iota3 + k * 48
                r = plsc.load_gather(inb, [idx0])
                g = plsc.load_gather(inb, [idx0 + 1])
                bl = plsc.load_gather(inb, [idx0 + 2])
                y = _b16(r) * w0 + _b16(g) * w1 + _b16(bl) * w2
                gout[pl.ds(k * 16, 16)] = y
                return (jnp.minimum(mn_, y), jnp.maximum(mx_, y))

            mn, mx = lax.fori_loop(0, _G // 16, ibody, (mn, mx))
            pltpu.async_copy(
                gout, gray_hbm.at[pl.ds(base_g + cur * _G, _G)], sout)

            @pl.when(cur + 2 < _NCH)
            def _(inb=inb, sin=sin, cur=cur):
                pltpu.async_copy(
                    x_hbm.at[pl.ds(base_in + (cur + 2) * _IN, _IN)], inb, sin)
        return (mn, mx)

    mn, mx = p1
    pltpu.make_async_copy(
        gout0, gray_hbm.at[pl.ds(base_g + (_NCH - 2) * _G, _G)],
        sem_out0).wait()
    pltpu.make_async_copy(
        gout1, gray_hbm.at[pl.ds(base_g + (_NCH - 1) * _G, _G)],
        sem_out1).wait()

    v16a[...] = jnp.full((16,), jnp.min(mn), jnp.float32)
    pltpu.sync_copy(v16a, gmn_hbm.at[pl.ds(wid * 16, 16)])
    v16b[...] = jnp.full((16,), jnp.max(mx), jnp.float32)
    pltpu.sync_copy(v16b, gmx_hbm.at[pl.ds(wid * 16, 16)])


def _sc_gray_stage(x_flat):
    mesh = plsc.VectorSubcoreMesh(core_axis_name="c", subcore_axis_name="s")
    kern = functools.partial(
        pl.kernel,
        mesh=mesh,
        compiler_params=pltpu.CompilerParams(needs_layout_passes=False),
        out_type=[
            jax.ShapeDtypeStruct((B * HW,), jnp.float32),
            jax.ShapeDtypeStruct((B * 16,), jnp.float32),
            jax.ShapeDtypeStruct((B * 16,), jnp.float32),
        ],
        scratch_types=[
            pltpu.VMEM((_IN,), jnp.float32),
            pltpu.VMEM((_IN,), jnp.float32),
            pltpu.VMEM((_G,), jnp.float32),
            pltpu.VMEM((_G,), jnp.float32),
            pltpu.VMEM((16,), jnp.float32),
            pltpu.VMEM((16,), jnp.float32),
            pltpu.SemaphoreType.DMA,
            pltpu.SemaphoreType.DMA,
            pltpu.SemaphoreType.DMA,
            pltpu.SemaphoreType.DMA,
        ],
    )(_sc_gray_body)
    return kern(x_flat)


def _sc_hist_body(gray_hbm, gmn_hbm, scale_hbm, hist_hbm,
                  g0, g1, subh, histv, mn_v, sc_v, sem0, sem1):
    wid = lax.axis_index("s") * _NC + lax.axis_index("c")
    base_g = wid * HW

    pltpu.sync_copy(gmn_hbm.at[pl.ds(wid * 16, 16)], mn_v)
    pltpu.sync_copy(scale_hbm.at[pl.ds(wid * 16, 16)], sc_v)
    gmn_v = mn_v[...]
    scale_v = sc_v[...]

    iota = lax.broadcasted_iota(jnp.int32, (16,), 0)
    lane_base = iota * NBINS
    ones16 = jnp.ones((16,), jnp.float32)
    zeros16 = jnp.zeros((16,), jnp.float32)

    def zbody(j, _):
        subh[pl.ds(j * 16, 16)] = zeros16
        return 0
    lax.fori_loop(0, (16 * NBINS) // 16, zbody, 0)

    gbs = (g0, g1)
    sems = (sem0, sem1)
    pltpu.async_copy(gray_hbm.at[pl.ds(base_g, _G)], g0, sem0)
    pltpu.async_copy(gray_hbm.at[pl.ds(base_g + _G, _G)], g1, sem1)

    @pl.loop(0, _NCH, step=2)
    def p2(ci):
        for b in range(2):
            cur = ci + b
            gb, sin = gbs[b], sems[b]
            pltpu.make_async_copy(
                gray_hbm.at[pl.ds(base_g + cur * _G, _G)], gb, sin).wait()

            def jbody(k, _, gb=gb):
                gy = gb[pl.ds(k * 16, 16)]
                t = (gy - gmn_v) * scale_v
                bi = jnp.minimum(t.astype(jnp.int32), NBINS - 1)
                plsc.addupdate_scatter(subh, [lane_base + bi], ones16)
                return 0
            lax.fori_loop(0, _G // 16, jbody, 0)

            @pl.when(cur + 2 < _NCH)
            def _(gb=gb, sin=sin, cur=cur):
                pltpu.async_copy(
                    gray_hbm.at[pl.ds(base_g + (cur + 2) * _G, _G)], gb, sin)

    def rbody(j, _):
        acc = subh[pl.ds(j * 16, 16)]
        for l in range(1, 16):
            acc = acc + subh[pl.ds(l * NBINS + j * 16, 16)]
        histv[pl.ds(j * 16, 16)] = acc
        return 0
    lax.fori_loop(0, NBINS // 16, rbody, 0)
    pltpu.sync_copy(histv, hist_hbm.at[pl.ds(wid * NBINS, NBINS)])


def _sc_hist_stage(gray_flat, gmn_flat, scale_flat):
    mesh = plsc.VectorSubcoreMesh(core_axis_name="c", subcore_axis_name="s")
    kern = functools.partial(
        pl.kernel,
        mesh=mesh,
        compiler_params=pltpu.CompilerParams(needs_layout_passes=False),
        out_type=jax.ShapeDtypeStruct((B * NBINS,), jnp.float32),
        scratch_types=[
            pltpu.VMEM((_G,), jnp.float32),
            pltpu.VMEM((_G,), jnp.float32),
            pltpu.VMEM((16 * NBINS,), jnp.float32),
            pltpu.VMEM((NBINS,), jnp.float32),
            pltpu.VMEM((16,), jnp.float32),
            pltpu.VMEM((16,), jnp.float32),
            pltpu.SemaphoreType.DMA,
            pltpu.SemaphoreType.DMA,
        ],
    )(_sc_hist_body)
    return kern(gray_flat, gmn_flat, scale_flat)


def _otsu_body(hist_ref, cen_ref, thr_ref):
    hist = hist_ref[...]
    centers = cen_ref[...]
    iota_i = lax.broadcasted_iota(jnp.int32, (B, NBINS), 1)
    wc = hist * centers

    def cumsum(x):
        k = 1
        while k < NBINS:
            pad = jnp.zeros((B, k), jnp.float32)
            x = x + jnp.concatenate([pad, x[:, :NBINS - k]], axis=1)
            k *= 2
        return x

    w1 = cumsum(hist)
    c1 = cumsum(wc)
    total_w = w1[:, NBINS - 1:]
    total_c = c1[:, NBINS - 1:]
    w2p = total_w - w1
    m1 = c1 / jnp.maximum(w1, 1e-12)
    m2 = (total_c - c1) / jnp.maximum(w2p, 1e-12)
    var12 = w1 * w2p * (m1 - m2) ** 2
    var12 = jnp.where(iota_i < NBINS - 1, var12, jnp.float32(-1.0))
    vmax = jnp.max(var12, axis=1, keepdims=True)
    idx = jnp.min(jnp.where(var12 == vmax, iota_i, NBINS), axis=1,
                  keepdims=True)
    thr = jnp.sum(jnp.where(iota_i == idx, centers, 0.0), axis=1,
                  keepdims=True)
    thr_ref[:, :, :] = jnp.broadcast_to(thr[:, None, :], (B, 8, 128))


def _otsu_stage(hist, centers):
    return pl.pallas_call(
        _otsu_body,
        out_shape=jax.ShapeDtypeStruct((B, 8, 128), jnp.float32),
    )(hist, centers)


def _binarize_body(gray_ref, thr_ref, out_ref):
    thr = thr_ref[0, 0, 0]
    out_ref[0] = jnp.where(gray_ref[0] > thr, jnp.float32(255.0),
                           jnp.float32(0.0))


def _binarize_stage(gray, thr):
    return pl.pallas_call(
        _binarize_body,
        grid=(B,),
        in_specs=[
            pl.BlockSpec((1, H, W), lambda i: (i, 0, 0)),
            pl.BlockSpec((1, 8, 128), lambda i: (i, 0, 0)),
        ],
        out_specs=pl.BlockSpec((1, H, W), lambda i: (i, 0, 0)),
        out_shape=jax.ShapeDtypeStruct((B, H, W), jnp.float32),
    )(gray, thr)


def kernel(inputs):
    x_flat = inputs.reshape(-1)
    gray_flat, gmn, gmx = _sc_gray_stage(x_flat)
    gmn32 = gmn.reshape(B, 16)[:, 0]
    gmx32 = gmx.reshape(B, 16)[:, 0]
    scale32 = NBINS / jnp.maximum(gmx32 - gmn32, 1e-12)
    scale_flat = jnp.broadcast_to(scale32[:, None], (B, 16)).reshape(-1)
    hist = _sc_hist_stage(gray_flat, gmn, scale_flat).reshape(B, NBINS)
    centers = gmn32[:, None] + (
        jnp.arange(NBINS, dtype=jnp.float32)[None, :] + 0.5) / scale32[:, None]
    thr = _otsu_stage(hist, centers)
    binary = _binarize_stage(gray_flat.reshape(B, H, W), thr)
    return jnp.repeat(binary[..., None], 3, axis=-1)


# speedup vs baseline: 30.2631x; 30.2631x over previous
import functools

import jax
import jax.numpy as jnp
from jax import lax
from jax.experimental import pallas as pl
from jax.experimental.pallas import tpu as pltpu
from jax.experimental.pallas import tpu_sc as plsc

B, H, W, C = 32, 512, 512, 3
NBINS = 256
HW = H * W
_W0, _W1, _W2 = 0.29882812, 0.5859375, 0.11376953

_NC, _NS = 2, 16
_G = 8192
_NCH = HW // _G


def _b16(v):
    u = lax.bitcast_convert_type(v, jnp.int32)
    lsb = lax.shift_right_logical(u, 16) & 1
    r = (u + 0x7FFF + lsb) & jnp.int32(-65536)
    return lax.bitcast_convert_type(r, jnp.float32)


def _gray_body(x_ref, gray_ref, mn_ref, mx_ref):
    r = x_ref[0, 0]
    g = x_ref[0, 1]
    bl = x_ref[0, 2]
    y = (_b16(r) * jnp.float32(_W0) + _b16(g) * jnp.float32(_W1)
         + _b16(bl) * jnp.float32(_W2))
    gray_ref[0] = y
    mn_ref[0] = jnp.broadcast_to(jnp.min(y), (8, 128))
    mx_ref[0] = jnp.broadcast_to(jnp.max(y), (8, 128))


def _gray_stage(xt):
    return pl.pallas_call(
        _gray_body,
        grid=(B,),
        in_specs=[pl.BlockSpec((1, C, H, W), lambda i: (i, 0, 0, 0))],
        out_specs=[
            pl.BlockSpec((1, H, W), lambda i: (i, 0, 0)),
            pl.BlockSpec((1, 8, 128), lambda i: (i, 0, 0)),
            pl.BlockSpec((1, 8, 128), lambda i: (i, 0, 0)),
        ],
        out_shape=[
            jax.ShapeDtypeStruct((B, H, W), jnp.float32),
            jax.ShapeDtypeStruct((B, 8, 128), jnp.float32),
            jax.ShapeDtypeStruct((B, 8, 128), jnp.float32),
        ],
    )(xt)


def _sc_hist_body(gray_hbm, gmn_hbm, scale_hbm, hist_hbm,
                  g0, g1, subh, histv, mn_v, sc_v, sem0, sem1):
    wid = lax.axis_index("s") * _NC + lax.axis_index("c")
    base_g = wid * HW

    pltpu.sync_copy(gmn_hbm.at[pl.ds(wid * 16, 16)], mn_v)
    pltpu.sync_copy(scale_hbm.at[pl.ds(wid * 16, 16)], sc_v)
    gmn_v = mn_v[...]
    scale_v = sc_v[...]

    iota = lax.broadcasted_iota(jnp.int32, (16,), 0)
    lane_base = iota * NBINS
    ones16 = jnp.ones((16,), jnp.float32)
    zeros16 = jnp.zeros((16,), jnp.float32)

    def zbody(j, _):
        subh[pl.ds(j * 16, 16)] = zeros16
        return 0
    lax.fori_loop(0, (16 * NBINS) // 16, zbody, 0)

    gbs = (g0, g1)
    sems = (sem0, sem1)
    pltpu.async_copy(gray_hbm.at[pl.ds(base_g, _G)], g0, sem0)
    pltpu.async_copy(gray_hbm.at[pl.ds(base_g + _G, _G)], g1, sem1)

    @pl.loop(0, _NCH, step=2)
    def p2(ci):
        for b in range(2):
            cur = ci + b
            gb, sin = gbs[b], sems[b]
            pltpu.make_async_copy(
                gray_hbm.at[pl.ds(base_g + cur * _G, _G)], gb, sin).wait()

            def jbody(k, _, gb=gb):
                gy = gb[pl.ds(k * 16, 16)]
                t = (gy - gmn_v) * scale_v
                bi = jnp.minimum(t.astype(jnp.int32), NBINS - 1)
                plsc.addupdate_scatter(subh, [lane_base + bi], ones16)
                return 0
            lax.fori_loop(0, _G // 16, jbody, 0)

            @pl.when(cur + 2 < _NCH)
            def _(gb=gb, sin=sin, cur=cur):
                pltpu.async_copy(
                    gray_hbm.at[pl.ds(base_g + (cur + 2) * _G, _G)], gb, sin)

    def rbody(j, _):
        acc = subh[pl.ds(j * 16, 16)]
        for l in range(1, 16):
            acc = acc + subh[pl.ds(l * NBINS + j * 16, 16)]
        histv[pl.ds(j * 16, 16)] = acc
        return 0
    lax.fori_loop(0, NBINS // 16, rbody, 0)
    pltpu.sync_copy(histv, hist_hbm.at[pl.ds(wid * NBINS, NBINS)])


def _sc_hist_stage(gray_flat, gmn_flat, scale_flat):
    mesh = plsc.VectorSubcoreMesh(core_axis_name="c", subcore_axis_name="s")
    kern = functools.partial(
        pl.kernel,
        mesh=mesh,
        compiler_params=pltpu.CompilerParams(needs_layout_passes=False),
        out_type=jax.ShapeDtypeStruct((B * NBINS,), jnp.float32),
        scratch_types=[
            pltpu.VMEM((_G,), jnp.float32),
            pltpu.VMEM((_G,), jnp.float32),
            pltpu.VMEM((16 * NBINS,), jnp.float32),
            pltpu.VMEM((NBINS,), jnp.float32),
            pltpu.VMEM((16,), jnp.float32),
            pltpu.VMEM((16,), jnp.float32),
            pltpu.SemaphoreType.DMA,
            pltpu.SemaphoreType.DMA,
        ],
    )(_sc_hist_body)
    return kern(gray_flat, gmn_flat, scale_flat)


def _otsu_body(hist_ref, cen_ref, thr_ref):
    hist = hist_ref[...]
    centers = cen_ref[...]
    iota_i = lax.broadcasted_iota(jnp.int32, (B, NBINS), 1)
    wc = hist * centers

    def cumsum(x):
        k = 1
        while k < NBINS:
            pad = jnp.zeros((B, k), jnp.float32)
            x = x + jnp.concatenate([pad, x[:, :NBINS - k]], axis=1)
            k *= 2
        return x

    w1 = cumsum(hist)
    c1 = cumsum(wc)
    total_w = w1[:, NBINS - 1:]
    total_c = c1[:, NBINS - 1:]
    w2p = total_w - w1
    m1 = c1 / jnp.maximum(w1, 1e-12)
    m2 = (total_c - c1) / jnp.maximum(w2p, 1e-12)
    var12 = w1 * w2p * (m1 - m2) ** 2
    var12 = jnp.where(iota_i < NBINS - 1, var12, jnp.float32(-1.0))
    vmax = jnp.max(var12, axis=1, keepdims=True)
    idx = jnp.min(jnp.where(var12 == vmax, iota_i, NBINS), axis=1,
                  keepdims=True)
    thr = jnp.sum(jnp.where(iota_i == idx, centers, 0.0), axis=1,
                  keepdims=True)
    thr_ref[:, :, :] = jnp.broadcast_to(thr[:, None, :], (B, 8, 128))


def _otsu_stage(hist, centers):
    return pl.pallas_call(
        _otsu_body,
        out_shape=jax.ShapeDtypeStruct((B, 8, 128), jnp.float32),
    )(hist, centers)


def _binarize_body(gray_ref, thr_ref, out_ref):
    thr = thr_ref[0, 0, 0]
    out_ref[0] = jnp.where(gray_ref[0] > thr, jnp.float32(255.0),
                           jnp.float32(0.0))


def _binarize_stage(gray, thr):
    return pl.pallas_call(
        _binarize_body,
        grid=(B,),
        in_specs=[
            pl.BlockSpec((1, H, W), lambda i: (i, 0, 0)),
            pl.BlockSpec((1, 8, 128), lambda i: (i, 0, 0)),
        ],
        out_specs=pl.BlockSpec((1, H, W), lambda i: (i, 0, 0)),
        out_shape=jax.ShapeDtypeStruct((B, H, W), jnp.float32),
    )(gray, thr)


def kernel(inputs):
    xt = jnp.transpose(inputs, (0, 3, 1, 2))
    gray, mn, mx = _gray_stage(xt)
    gmn32 = mn[:, 0, 0]
    gmx32 = mx[:, 0, 0]
    scale32 = NBINS / jnp.maximum(gmx32 - gmn32, 1e-12)
    gmn_flat = jnp.broadcast_to(gmn32[:, None], (B, 16)).reshape(-1)
    scale_flat = jnp.broadcast_to(scale32[:, None], (B, 16)).reshape(-1)
    grayv = gray.reshape(B, H // 8, 8, W // 128, 128).transpose(
        0, 1, 3, 2, 4).reshape(-1)
    hist = _sc_hist_stage(grayv, gmn_flat, scale_flat)
    hist = hist.reshape(B, NBINS)
    centers = gmn32[:, None] + (
        jnp.arange(NBINS, dtype=jnp.float32)[None, :] + 0.5) / scale32[:, None]
    thr = _otsu_stage(hist, centers)
    binary = _binarize_stage(gray, thr)
    return jnp.repeat(binary[..., None], 3, axis=-1)


# speedup vs baseline: 32.1908x; 1.0637x over previous
import functools

import jax
import jax.numpy as jnp
from jax import lax
from jax.experimental import pallas as pl
from jax.experimental.pallas import tpu as pltpu
from jax.experimental.pallas import tpu_sc as plsc

B, H, W, C = 32, 512, 512, 3
NBINS = 256
HW = H * W
_W0, _W1, _W2 = 0.29882812, 0.5859375, 0.11376953

_NC, _NS = 2, 16
_G = 8192
_NCH = HW // _G


def _b16(v):
    u = lax.bitcast_convert_type(v, jnp.int32)
    lsb = lax.shift_right_logical(u, 16) & 1
    r = (u + 0x7FFF + lsb) & jnp.int32(-65536)
    return lax.bitcast_convert_type(r, jnp.float32)


def _gray_body(x_ref, gray_ref, mn_ref, mx_ref):
    r = x_ref[0, 0]
    g = x_ref[0, 1]
    bl = x_ref[0, 2]
    y = (_b16(r) * jnp.float32(_W0) + _b16(g) * jnp.float32(_W1)
         + _b16(bl) * jnp.float32(_W2))
    gray_ref[0] = y
    mn_ref[0] = jnp.broadcast_to(jnp.min(y), (8, 128))
    mx_ref[0] = jnp.broadcast_to(jnp.max(y), (8, 128))


def _gray_stage(xt):
    return pl.pallas_call(
        _gray_body,
        grid=(B,),
        in_specs=[pl.BlockSpec((1, C, H, W), lambda i: (i, 0, 0, 0))],
        out_specs=[
            pl.BlockSpec((1, H, W), lambda i: (i, 0, 0)),
            pl.BlockSpec((1, 8, 128), lambda i: (i, 0, 0)),
            pl.BlockSpec((1, 8, 128), lambda i: (i, 0, 0)),
        ],
        out_shape=[
            jax.ShapeDtypeStruct((B, H, W), jnp.float32),
            jax.ShapeDtypeStruct((B, 8, 128), jnp.float32),
            jax.ShapeDtypeStruct((B, 8, 128), jnp.float32),
        ],
    )(xt)


def _sc_hist_body(gray_hbm, gmn_hbm, scale_hbm, hist_hbm,
                  g0, g1, subh, histv, mn_v, sc_v, sem0, sem1):
    wid = lax.axis_index("s") * _NC + lax.axis_index("c")
    base_g = wid * HW

    pltpu.sync_copy(gmn_hbm.at[pl.ds(wid * 16, 16)], mn_v)
    pltpu.sync_copy(scale_hbm.at[pl.ds(wid * 16, 16)], sc_v)
    gmn_v = mn_v[...]
    scale_v = sc_v[...]

    iota = lax.broadcasted_iota(jnp.int32, (16,), 0)
    lane_base = iota * NBINS
    ones16 = jnp.ones((16,), jnp.float32)
    zeros16 = jnp.zeros((16,), jnp.float32)

    def zbody(j, _):
        subh[pl.ds(j * 16, 16)] = zeros16
        return 0
    lax.fori_loop(0, (16 * NBINS) // 16, zbody, 0)

    gbs = (g0, g1)
    sems = (sem0, sem1)
    pltpu.async_copy(gray_hbm.at[pl.ds(base_g, _G)], g0, sem0)
    pltpu.async_copy(gray_hbm.at[pl.ds(base_g + _G, _G)], g1, sem1)

    @pl.loop(0, _NCH, step=2)
    def p2(ci):
        for b in range(2):
            cur = ci + b
            gb, sin = gbs[b], sems[b]
            pltpu.make_async_copy(
                gray_hbm.at[pl.ds(base_g + cur * _G, _G)], gb, sin).wait()

            def jbody(k, _, gb=gb):
                for u in range(8):
                    gy = gb[pl.ds(k * 128 + u * 16, 16)]
                    t = (gy - gmn_v) * scale_v
                    bi = jnp.minimum(t.astype(jnp.int32), NBINS - 1)
                    plsc.addupdate_scatter(subh, [lane_base + bi], ones16)
                return 0
            lax.fori_loop(0, _G // 128, jbody, 0)

            @pl.when(cur + 2 < _NCH)
            def _(gb=gb, sin=sin, cur=cur):
                pltpu.async_copy(
                    gray_hbm.at[pl.ds(base_g + (cur + 2) * _G, _G)], gb, sin)

    def rbody(j, _):
        acc = subh[pl.ds(j * 16, 16)]
        for l in range(1, 16):
            acc = acc + subh[pl.ds(l * NBINS + j * 16, 16)]
        histv[pl.ds(j * 16, 16)] = acc
        return 0
    lax.fori_loop(0, NBINS // 16, rbody, 0)
    pltpu.sync_copy(histv, hist_hbm.at[pl.ds(wid * NBINS, NBINS)])


def _sc_hist_stage(gray_flat, gmn_flat, scale_flat):
    mesh = plsc.VectorSubcoreMesh(core_axis_name="c", subcore_axis_name="s")
    kern = functools.partial(
        pl.kernel,
        mesh=mesh,
        compiler_params=pltpu.CompilerParams(needs_layout_passes=False),
        out_type=jax.ShapeDtypeStruct((B * NBINS,), jnp.float32),
        scratch_types=[
            pltpu.VMEM((_G,), jnp.float32),
            pltpu.VMEM((_G,), jnp.float32),
            pltpu.VMEM((16 * NBINS,), jnp.float32),
            pltpu.VMEM((NBINS,), jnp.float32),
            pltpu.VMEM((16,), jnp.float32),
            pltpu.VMEM((16,), jnp.float32),
            pltpu.SemaphoreType.DMA,
            pltpu.SemaphoreType.DMA,
        ],
    )(_sc_hist_body)
    return kern(gray_flat, gmn_flat, scale_flat)


def _otsu_body(hist_ref, cen_ref, thr_ref):
    hist = hist_ref[...]
    centers = cen_ref[...]
    iota_i = lax.broadcasted_iota(jnp.int32, (B, NBINS), 1)
    wc = hist * centers

    def cumsum(x):
        k = 1
        while k < NBINS:
            pad = jnp.zeros((B, k), jnp.float32)
            x = x + jnp.concatenate([pad, x[:, :NBINS - k]], axis=1)
            k *= 2
        return x

    w1 = cumsum(hist)
    c1 = cumsum(wc)
    total_w = w1[:, NBINS - 1:]
    total_c = c1[:, NBINS - 1:]
    w2p = total_w - w1
    m1 = c1 / jnp.maximum(w1, 1e-12)
    m2 = (total_c - c1) / jnp.maximum(w2p, 1e-12)
    var12 = w1 * w2p * (m1 - m2) ** 2
    var12 = jnp.where(iota_i < NBINS - 1, var12, jnp.float32(-1.0))
    vmax = jnp.max(var12, axis=1, keepdims=True)
    idx = jnp.min(jnp.where(var12 == vmax, iota_i, NBINS), axis=1,
                  keepdims=True)
    thr = jnp.sum(jnp.where(iota_i == idx, centers, 0.0), axis=1,
                  keepdims=True)
    thr_ref[:, :, :] = jnp.broadcast_to(thr[:, None, :], (B, 8, 128))


def _otsu_stage(hist, centers):
    return pl.pallas_call(
        _otsu_body,
        out_shape=jax.ShapeDtypeStruct((B, 8, 128), jnp.float32),
    )(hist, centers)


def _binarize_body(gray_ref, thr_ref, out_ref):
    thr = thr_ref[0, 0, 0]
    y = jnp.where(gray_ref[0] > thr, jnp.float32(255.0), jnp.float32(0.0))
    out_ref[0, 0] = y
    out_ref[0, 1] = y
    out_ref[0, 2] = y


def _binarize_stage(gray, thr):
    return pl.pallas_call(
        _binarize_body,
        grid=(B,),
        in_specs=[
            pl.BlockSpec((1, H, W), lambda i: (i, 0, 0)),
            pl.BlockSpec((1, 8, 128), lambda i: (i, 0, 0)),
        ],
        out_specs=pl.BlockSpec((1, C, H, W), lambda i: (i, 0, 0, 0)),
        out_shape=jax.ShapeDtypeStruct((B, C, H, W), jnp.float32),
    )(gray, thr)


def kernel(inputs):
    xt = jnp.transpose(inputs, (0, 3, 1, 2))
    gray, mn, mx = _gray_stage(xt)
    gmn32 = mn[:, 0, 0]
    gmx32 = mx[:, 0, 0]
    scale32 = NBINS / jnp.maximum(gmx32 - gmn32, 1e-12)
    gmn_flat = jnp.broadcast_to(gmn32[:, None], (B, 16)).reshape(-1)
    scale_flat = jnp.broadcast_to(scale32[:, None], (B, 16)).reshape(-1)
    grayv = gray.reshape(B, H // 8, 8, W // 128, 128).transpose(
        0, 1, 3, 2, 4).reshape(-1)
    hist = _sc_hist_stage(grayv, gmn_flat, scale_flat)
    hist = hist.reshape(B, NBINS)
    centers = gmn32[:, None] + (
        jnp.arange(NBINS, dtype=jnp.float32)[None, :] + 0.5) / scale32[:, None]
    thr = _otsu_stage(hist, centers)
    out4 = _binarize_stage(gray, thr)
    return jnp.transpose(out4, (0, 2, 3, 1))
